# baseline (speedup 1.0000x reference)
import functools

import jax
import jax.numpy as jnp
from jax import lax
from jax.experimental import pallas as pl
from jax.experimental.pallas import tpu as pltpu
from jax.experimental.pallas import tpu_sc as plsc

VOCAB = 100000
EMB = 64
BATCH = 1024
HIST = 50

NIDX = BATCH * HIST
_NC, _NS, _L = 2, 16, 16
_PER_TILE = NIDX // _NS

_MV_BLK = 10000


def _matvec_body(emb_ref, wt_ref, out_ref):
    out_ref[...] = lax.dot_general(
        emb_ref[...], wt_ref[...], (((1,), (0,)), ((), ())),
        preferred_element_type=jnp.float32,
    ) * (1.0 / HIST)


def _make_matvec(blk):
    return pl.pallas_call(
        _matvec_body,
        grid=(VOCAB // blk,),
        in_specs=[
            pl.BlockSpec((blk, EMB), lambda k: (k, 0)),
            pl.BlockSpec((EMB, 1), lambda k: (0, 0)),
        ],
        out_specs=pl.BlockSpec((blk, 1), lambda k: (k, 0)),
        out_shape=jax.ShapeDtypeStruct((VOCAB, 1), jnp.float32),
    )


def _matvec_tt_body(w_ref, embt_ref, out_ref):
    res = lax.dot_general(
        w_ref[...], embt_ref[...], (((1,), (0,)), ((), ())),
        preferred_element_type=jnp.float32,
    ) * (1.0 / HIST)
    out_ref[...] = res.reshape(1, 1, res.shape[-1])


VOCAB_PAD = 100096


def _make_matvec_tt(blk):
    return pl.pallas_call(
        _matvec_tt_body,
        grid=(VOCAB_PAD // blk,),
        in_specs=[
            pl.BlockSpec((1, EMB), lambda k: (0, 0)),
            pl.BlockSpec((EMB, blk), lambda k: (0, k)),
        ],
        out_specs=pl.BlockSpec((1, 1, blk), lambda k: (k, 0, 0)),
        out_shape=jax.ShapeDtypeStruct((VOCAB_PAD // blk, 1, blk), jnp.float32),
    )


def _matvec_t_body(w_ref, emb_ref, out_ref):
    res = lax.dot_general(
        w_ref[...], emb_ref[...], (((1,), (1,)), ((), ())),
        preferred_element_type=jnp.float32,
    ) * (1.0 / HIST)
    out_ref[...] = res.reshape(1, 1, res.shape[-1])


def _make_matvec_t(blk):
    return pl.pallas_call(
        _matvec_t_body,
        grid=(VOCAB // blk,),
        in_specs=[
            pl.BlockSpec((1, EMB), lambda k: (0, 0)),
            pl.BlockSpec((blk, EMB), lambda k: (k, 0)),
        ],
        out_specs=pl.BlockSpec((1, 1, blk), lambda k: (k, 0, 0)),
        out_shape=jax.ShapeDtypeStruct((VOCAB // blk, 1, blk), jnp.float32),
    )


def _gather_body(tbl_hbm, idx_hbm, out_hbm, shared, idx_v, out_v, sem):
    c = lax.axis_index("c")
    s = lax.axis_index("s")
    base = s * _PER_TILE

    @pl.when(s == 0)
    def _():
        pltpu.sync_copy(tbl_hbm.at[c], shared)

    plsc.subcore_barrier()
    pltpu.sync_copy(idx_hbm.at[pl.ds(base, _PER_TILE)], idx_v)
    pltpu.async_copy(shared.at[idx_v], out_v, sem).wait()
    pltpu.sync_copy(out_v, out_hbm.at[c, pl.ds(base, _PER_TILE)])


@functools.cache
def _make_gather():
    return pl.kernel(
        _gather_body,
        out_type=jax.ShapeDtypeStruct((2, NIDX), jnp.float32),
        mesh=plsc.VectorSubcoreMesh(
            core_axis_name="c", subcore_axis_name="s",
            num_cores=_NC, num_subcores=_NS,
        ),
        scratch_types=(
            pltpu.VMEM_SHARED((VOCAB_PAD,), jnp.float32),
            pltpu.VMEM((_PER_TILE,), jnp.int32),
            pltpu.VMEM((_PER_TILE,), jnp.float32),
            pltpu.SemaphoreType.DMA,
        ),
        compiler_params=pltpu.CompilerParams(use_tc_tiling_on_sc=False),
    )


def _finish_body(x_ref, g_ref, w_ref, b_ref, out_ref):
    xt = x_ref[...]
    wt = w_ref[...]
    gsum = jnp.sum(g_ref[...], axis=0, keepdims=True)
    wsum = wt[0:1, :]
    for j in range(1, HIST):
        xj = xt[j:j + 1, :]
        dup = xt[0:1, :] == xj
        for jp in range(1, j):
            dup = dup | (xt[jp:jp + 1, :] == xj)
        wsum = wsum + jnp.where(dup, 0.0, wt[j:j + 1, :])
    z = gsum + wsum + b_ref[0, 0]
    out_ref[...] = 1.0 / (1.0 + jnp.exp(-z))


_finish = pl.pallas_call(
    _finish_body,
    out_shape=jax.ShapeDtypeStruct((1, BATCH), jnp.float32),
)


def kernel(x, emb_table, W, b):
    xt = x.astype(jnp.int32).T
    embt = jnp.pad(emb_table.T, ((0, 0), (0, VOCAB_PAD - VOCAB)))
    t = _make_matvec_tt(50048)(W[:, :EMB], embt)
    wv_pad = jnp.pad(W[0, EMB:], (0, VOCAB_PAD - VOCAB))
    tbl = jnp.concatenate(
        [t.reshape(1, VOCAB_PAD), wv_pad.reshape(1, VOCAB_PAD)], 0)
    gw = _make_gather()(tbl, xt.reshape(-1))
    y = _finish(xt, gw[0].reshape(HIST, BATCH), gw[1].reshape(HIST, BATCH),
                b.reshape(1, 1))
    return y.reshape(BATCH, 1)


# speedup vs baseline: 1.1678x; 1.1678x over previous
import functools

import jax
import jax.numpy as jnp
from jax import lax
from jax.experimental import pallas as pl
from jax.experimental.pallas import tpu as pltpu
from jax.experimental.pallas import tpu_sc as plsc

VOCAB = 100000
EMB = 64
BATCH = 1024
HIST = 50

NIDX = BATCH * HIST
_NC, _NS, _L = 2, 16, 16
_PER_TILE = NIDX // _NS

_MV_BLK = 10000


def _matvec_body(emb_ref, wt_ref, out_ref):
    out_ref[...] = lax.dot_general(
        emb_ref[...], wt_ref[...], (((1,), (0,)), ((), ())),
        preferred_element_type=jnp.float32,
    ) * (1.0 / HIST)


def _make_matvec(blk):
    return pl.pallas_call(
        _matvec_body,
        grid=(VOCAB // blk,),
        in_specs=[
            pl.BlockSpec((blk, EMB), lambda k: (k, 0)),
            pl.BlockSpec((EMB, 1), lambda k: (0, 0)),
        ],
        out_specs=pl.BlockSpec((blk, 1), lambda k: (k, 0)),
        out_shape=jax.ShapeDtypeStruct((VOCAB, 1), jnp.float32),
    )


def _matvec_tt_body(w_ref, embt_ref, out_ref):
    res = lax.dot_general(
        w_ref[...], embt_ref[...], (((1,), (0,)), ((), ())),
        preferred_element_type=jnp.float32,
    ) * (1.0 / HIST)
    out_ref[...] = res.reshape(1, 1, res.shape[-1])


VOCAB_PAD = 100096


def _make_matvec_tt(blk):
    return pl.pallas_call(
        _matvec_tt_body,
        grid=(VOCAB_PAD // blk,),
        in_specs=[
            pl.BlockSpec((1, EMB), lambda k: (0, 0)),
            pl.BlockSpec((EMB, blk), lambda k: (0, k)),
        ],
        out_specs=pl.BlockSpec((1, 1, blk), lambda k: (k, 0, 0)),
        out_shape=jax.ShapeDtypeStruct((VOCAB_PAD // blk, 1, blk), jnp.float32),
    )


def _matvec_t_body(w_ref, emb_ref, out_ref):
    res = lax.dot_general(
        w_ref[...], emb_ref[...], (((1,), (1,)), ((), ())),
        preferred_element_type=jnp.float32,
    ) * (1.0 / HIST)
    out_ref[...] = res.reshape(1, 1, res.shape[-1])


def _make_matvec_t(blk):
    return pl.pallas_call(
        _matvec_t_body,
        grid=(VOCAB // blk,),
        in_specs=[
            pl.BlockSpec((1, EMB), lambda k: (0, 0)),
            pl.BlockSpec((blk, EMB), lambda k: (k, 0)),
        ],
        out_specs=pl.BlockSpec((1, 1, blk), lambda k: (k, 0, 0)),
        out_shape=jax.ShapeDtypeStruct((VOCAB // blk, 1, blk), jnp.float32),
    )


def _gather_body(tbl_hbm, idx_hbm, out_hbm, shared, idx_v, out_v, sem):
    c = lax.axis_index("c")
    s = lax.axis_index("s")
    base = s * _PER_TILE

    @pl.when(s == 0)
    def _():
        pltpu.sync_copy(tbl_hbm.at[c], shared)

    plsc.subcore_barrier()
    pltpu.sync_copy(idx_hbm.at[pl.ds(base, _PER_TILE)], idx_v)
    pltpu.async_copy(shared.at[idx_v], out_v, sem).wait()
    pltpu.sync_copy(out_v, out_hbm.at[c, pl.ds(base, _PER_TILE)])


_B_PER_W = BATCH // (_NC * _NS)
_I_PER_W = _B_PER_W * HIST


def _fused_body(tbl_hbm, xb_hbm, y_hbm, shared, idx_v, g_v, w_v, y_v, sem):
    c = lax.axis_index("c")
    s = lax.axis_index("s")
    wid = c * _NS + s

    @pl.when(s == 0)
    def _():
        pltpu.sync_copy(tbl_hbm.at[0], shared.at[0])

    @pl.when(s == 1)
    def _():
        pltpu.sync_copy(tbl_hbm.at[1], shared.at[1])

    pltpu.sync_copy(xb_hbm.at[pl.ds(wid * _I_PER_W, _I_PER_W)], idx_v)
    plsc.subcore_barrier()
    pltpu.async_copy(shared.at[0].at[idx_v], g_v, sem).wait()
    pltpu.async_copy(shared.at[1].at[idx_v], w_v, sem).wait()

    def half(h, carry):
        base = h * _L
        gsum = g_v[pl.ds(base, _L)]
        wsum = w_v[pl.ds(base, _L)]
        for j in range(1, HIST):
            off = j * _B_PER_W + base
            gsum = gsum + g_v[pl.ds(off, _L)]
            xj = idx_v[pl.ds(off, _L)]
            dup = idx_v[pl.ds(base, _L)] == xj
            for jp in range(1, j):
                dup = dup | (idx_v[pl.ds(jp * _B_PER_W + base, _L)] == xj)
            wsum = wsum + jnp.where(dup, 0.0, w_v[pl.ds(off, _L)])
        z = gsum + wsum
        y_v[pl.ds(base, _L)] = 1.0 / (1.0 + jnp.exp(-z))
        return carry

    lax.fori_loop(0, _B_PER_W // _L, half, 0)
    pltpu.sync_copy(y_v, y_hbm.at[pl.ds(wid * _B_PER_W, _B_PER_W)])


@functools.cache
def _make_fused():
    return pl.kernel(
        _fused_body,
        out_type=jax.ShapeDtypeStruct((BATCH,), jnp.float32),
        mesh=plsc.VectorSubcoreMesh(
            core_axis_name="c", subcore_axis_name="s",
            num_cores=_NC, num_subcores=_NS,
        ),
        scratch_types=(
            pltpu.VMEM_SHARED((2, VOCAB_PAD), jnp.float32),
            pltpu.VMEM((_I_PER_W,), jnp.int32),
            pltpu.VMEM((_I_PER_W,), jnp.float32),
            pltpu.VMEM((_I_PER_W,), jnp.float32),
            pltpu.VMEM((_B_PER_W,), jnp.float32),
            pltpu.SemaphoreType.DMA,
        ),
        compiler_params=pltpu.CompilerParams(use_tc_tiling_on_sc=False),
    )


@functools.cache
def _make_gather():
    return pl.kernel(
        _gather_body,
        out_type=jax.ShapeDtypeStruct((2, NIDX), jnp.float32),
        mesh=plsc.VectorSubcoreMesh(
            core_axis_name="c", subcore_axis_name="s",
            num_cores=_NC, num_subcores=_NS,
        ),
        scratch_types=(
            pltpu.VMEM_SHARED((VOCAB_PAD,), jnp.float32),
            pltpu.VMEM((_PER_TILE,), jnp.int32),
            pltpu.VMEM((_PER_TILE,), jnp.float32),
            pltpu.SemaphoreType.DMA,
        ),
        compiler_params=pltpu.CompilerParams(use_tc_tiling_on_sc=False),
    )


def _finish_body(x_ref, g_ref, w_ref, b_ref, out_ref):
    xt = x_ref[...]
    wt = w_ref[...]
    gsum = jnp.sum(g_ref[...], axis=0, keepdims=True)
    wsum = wt[0:1, :]
    for j in range(1, HIST):
        xj = xt[j:j + 1, :]
        dup = xt[0:1, :] == xj
        for jp in range(1, j):
            dup = dup | (xt[jp:jp + 1, :] == xj)
        wsum = wsum + jnp.where(dup, 0.0, wt[j:j + 1, :])
    z = gsum + wsum + b_ref[0, 0]
    out_ref[...] = 1.0 / (1.0 + jnp.exp(-z))


_finish = pl.pallas_call(
    _finish_body,
    out_shape=jax.ShapeDtypeStruct((1, BATCH), jnp.float32),
)


def kernel(x, emb_table, W, b):
    xb = (x.astype(jnp.int32)
          .reshape(_NC * _NS, _B_PER_W, HIST)
          .transpose(0, 2, 1).reshape(-1))
    embt = jnp.pad(emb_table.T, ((0, 0), (0, VOCAB_PAD - VOCAB)))
    t = _make_matvec_tt(50048)(W[:, :EMB], embt)
    wv_pad = jnp.pad(W[0, EMB:], (0, VOCAB_PAD - VOCAB))
    tbl = jnp.concatenate(
        [t.reshape(1, VOCAB_PAD) + b[0] * (1.0 / HIST),
         wv_pad.reshape(1, VOCAB_PAD)], 0)
    y = _make_fused()(tbl, xb)
    return y.reshape(BATCH, 1)
